# baseline (speedup 1.0000x reference)
import functools

import jax
import jax.numpy as jnp
from jax import lax
from jax.experimental import pallas as pl
from jax.experimental.pallas import tpu as pltpu
from jax.experimental.pallas import tpu_sc as plsc

_K = 1024
_M = 1024
_B = 8
_C = 384
_N = 1024
_CPW = 16
_AW = _C // _CPW
_L = 16


def _argmin_body(x_ref, cb_ref, cbsq_ref, xsq_ref, idx_ref, cbt_ref):
    xb = x_ref[0]
    cb = cb_ref[...]
    crossT = lax.dot_general(cb, xb, (((1,), (0,)), ((), ())),
                             preferred_element_type=jnp.float32)
    xsq = xsq_ref[0]
    t = (xsq + cbsq_ref[...]) - 2.0 * crossT
    dist = jnp.sqrt(jnp.clip(t, 0.0, None))
    minv = jnp.min(dist, axis=0, keepdims=True)
    kiota = lax.broadcasted_iota(jnp.int32, (_K, _N), 0)
    masked = jnp.where(dist == minv, kiota, _K)
    idx_ref[0] = jnp.min(masked, axis=0, keepdims=True)

    @pl.when(pl.program_id(0) == 0)
    def _():
        cbt_ref[...] = cb.T


def _sc_gather(cbt_flat, idx_flat):
    mesh = plsc.VectorSubcoreMesh(core_axis_name="c", subcore_axis_name="s")

    @functools.partial(
        pl.kernel,
        mesh=mesh,
        out_type=jax.ShapeDtypeStruct((_B * _C * _N,), jnp.float32),
        compiler_params=pltpu.CompilerParams(needs_layout_passes=False),
        scratch_types=[
            pltpu.VMEM((_CPW * _K,), jnp.float32),
            pltpu.VMEM((_B * _N,), jnp.int32),
            pltpu.VMEM((2 * _CPW * _N,), jnp.float32),
            pltpu.SemaphoreType.DMA,
            pltpu.SemaphoreType.DMA,
        ],
    )
    def k(cbt_hbm, idx_hbm, q_hbm, tab_v, idx_v, out_v, sem0, sem1):
        wid = lax.axis_index("s") * 2 + lax.axis_index("c")

        @pl.when(wid < _AW)
        def _():
            cs = wid * _CPW
            pltpu.sync_copy(cbt_hbm.at[pl.ds(cs * _K, _CPW * _K)], tab_v)
            pltpu.sync_copy(idx_hbm, idx_v)
            sems = (sem0, sem1)
            descs = [None, None]
            for b in range(_B):
                slot = b % 2
                if descs[slot] is not None:
                    descs[slot].wait()
                ob = slot * _CPW * _N

                @plsc.parallel_loop(0, _N // _L, unroll=4)
                def body(i):
                    idxs = idx_v[pl.ds(b * _N + i * _L, _L)]
                    for r in range(_CPW):
                        out_v[pl.ds(ob + r * _N + i * _L, _L)] = (
                            plsc.load_gather(tab_v, [idxs + r * _K]))

                descs[slot] = pltpu.async_copy(
                    out_v.at[pl.ds(ob, _CPW * _N)],
                    q_hbm.at[pl.ds(b * _C * _N + cs * _N, _CPW * _N)],
                    sems[slot])
            descs[0].wait()
            descs[1].wait()

    return k(cbt_flat, idx_flat)


def kernel(x, codebook):
    b, c, h, w = x.shape
    n = h * w
    x3 = x.reshape(b, c, n)
    cb_sq = jnp.sum(codebook * codebook, axis=-1)
    cbsq2 = cb_sq.reshape(_K, 1)
    flat = jnp.transpose(x, (0, 2, 3, 1)).reshape(b, n, c).astype(jnp.float32)
    x_sq = jnp.sum(flat * flat, axis=-1)
    xsq3 = x_sq.reshape(b, 1, n)
    idx, cbt = pl.pallas_call(
        _argmin_body,
        grid=(b,),
        in_specs=[
            pl.BlockSpec((1, c, n), lambda i: (i, 0, 0)),
            pl.BlockSpec((_K, c), lambda i: (0, 0)),
            pl.BlockSpec((_K, 1), lambda i: (0, 0)),
            pl.BlockSpec((1, 1, n), lambda i: (i, 0, 0)),
        ],
        out_specs=[
            pl.BlockSpec((1, 1, n), lambda i: (i, 0, 0)),
            pl.BlockSpec((_C, _K), lambda i: (0, 0)),
        ],
        out_shape=[
            jax.ShapeDtypeStruct((b, 1, n), jnp.int32),
            jax.ShapeDtypeStruct((_C, _K), jnp.float32),
        ],
    )(x3, codebook, cbsq2, xsq3)
    idx2 = idx.reshape(b, n)
    qf = _sc_gather(cbt.reshape(-1), idx2.reshape(-1))
    quantized = qf.reshape(b, c, h, w)
    embed_index = idx2.reshape(b, h, w)
    loss = jnp.array([0.0], dtype=jnp.float32)
    return (quantized, embed_index, loss)


# speedup vs baseline: 1.0219x; 1.0219x over previous
import functools

import jax
import jax.numpy as jnp
from jax import lax
from jax.experimental import pallas as pl
from jax.experimental.pallas import tpu as pltpu
from jax.experimental.pallas import tpu_sc as plsc

_K = 1024
_M = 1024
_B = 8
_C = 384
_N = 1024
_CPW = 12
_AW = _C // _CPW
_L = 16


def _argmin_body(x_ref, cb_ref, cbsq_ref, xsq_ref, idx_ref, cbt_ref):
    xb = x_ref[0]
    cb = cb_ref[...]
    crossT = lax.dot_general(cb, xb, (((1,), (0,)), ((), ())),
                             preferred_element_type=jnp.float32)
    xsq = xsq_ref[0]
    t = (xsq + cbsq_ref[...]) - 2.0 * crossT
    dist = jnp.sqrt(jnp.clip(t, 0.0, None))
    minv = jnp.min(dist, axis=0, keepdims=True)
    kiota = lax.broadcasted_iota(jnp.int32, (_K, _N), 0)
    masked = jnp.where(dist == minv, kiota, _K)
    idx_ref[0] = jnp.min(masked, axis=0, keepdims=True)

    @pl.when(pl.program_id(0) == 0)
    def _():
        cbt_ref[...] = cb.T


def _sc_gather(cbt_flat, idx_flat):
    mesh = plsc.VectorSubcoreMesh(core_axis_name="c", subcore_axis_name="s")

    @functools.partial(
        pl.kernel,
        mesh=mesh,
        out_type=jax.ShapeDtypeStruct((_B * _C * _N,), jnp.float32),
        compiler_params=pltpu.CompilerParams(needs_layout_passes=False),
        scratch_types=[
            pltpu.VMEM((_CPW * _K,), jnp.float32),
            pltpu.VMEM((_B * _N,), jnp.int32),
            pltpu.VMEM((2 * _CPW * _N,), jnp.float32),
            pltpu.SemaphoreType.DMA,
            pltpu.SemaphoreType.DMA,
        ],
    )
    def k(cbt_hbm, idx_hbm, q_hbm, tab_v, idx_v, out_v, sem0, sem1):
        wid = lax.axis_index("s") * 2 + lax.axis_index("c")

        @pl.when(wid < _AW)
        def _():
            cs = wid * _CPW
            pltpu.sync_copy(cbt_hbm.at[pl.ds(cs * _K, _CPW * _K)], tab_v)
            pltpu.sync_copy(idx_hbm, idx_v)
            sems = (sem0, sem1)
            descs = [None, None]
            for b in range(_B):
                slot = b % 2
                if descs[slot] is not None:
                    descs[slot].wait()
                ob = slot * _CPW * _N

                @plsc.parallel_loop(0, _N // _L, unroll=8)
                def body(i):
                    idxs = idx_v[pl.ds(b * _N + i * _L, _L)]
                    for r in range(_CPW):
                        out_v[pl.ds(ob + r * _N + i * _L, _L)] = (
                            plsc.load_gather(tab_v.at[pl.ds(r * _K, _K)],
                                             [idxs]))

                descs[slot] = pltpu.async_copy(
                    out_v.at[pl.ds(ob, _CPW * _N)],
                    q_hbm.at[pl.ds(b * _C * _N + cs * _N, _CPW * _N)],
                    sems[slot])
            descs[0].wait()
            descs[1].wait()

    return k(cbt_flat, idx_flat)


def kernel(x, codebook):
    b, c, h, w = x.shape
    n = h * w
    x3 = x.reshape(b, c, n)
    cb_sq = jnp.sum(codebook * codebook, axis=-1)
    cbsq2 = cb_sq.reshape(_K, 1)
    flat = jnp.transpose(x, (0, 2, 3, 1)).reshape(b, n, c).astype(jnp.float32)
    x_sq = jnp.sum(flat * flat, axis=-1)
    xsq3 = x_sq.reshape(b, 1, n)
    idx, cbt = pl.pallas_call(
        _argmin_body,
        grid=(b,),
        in_specs=[
            pl.BlockSpec((1, c, n), lambda i: (i, 0, 0)),
            pl.BlockSpec((_K, c), lambda i: (0, 0)),
            pl.BlockSpec((_K, 1), lambda i: (0, 0)),
            pl.BlockSpec((1, 1, n), lambda i: (i, 0, 0)),
        ],
        out_specs=[
            pl.BlockSpec((1, 1, n), lambda i: (i, 0, 0)),
            pl.BlockSpec((_C, _K), lambda i: (0, 0)),
        ],
        out_shape=[
            jax.ShapeDtypeStruct((b, 1, n), jnp.int32),
            jax.ShapeDtypeStruct((_C, _K), jnp.float32),
        ],
    )(x3, codebook, cbsq2, xsq3)
    idx2 = idx.reshape(b, n)
    qf = _sc_gather(cbt.reshape(-1), idx2.reshape(-1))
    quantized = qf.reshape(b, c, h, w)
    embed_index = idx2.reshape(b, h, w)
    loss = jnp.array([0.0], dtype=jnp.float32)
    return (quantized, embed_index, loss)


# speedup vs baseline: 1.0491x; 1.0266x over previous
import functools

import jax
import jax.numpy as jnp
from jax import lax
from jax.experimental import pallas as pl
from jax.experimental.pallas import tpu as pltpu
from jax.experimental.pallas import tpu_sc as plsc

_K = 1024
_M = 1024
_B = 8
_C = 384
_N = 1024
_CPW = 12
_AW = _C // _CPW
_L = 16


def _argmin_body(x_ref, cb_ref, cbsq_ref, xsq_ref, idx_ref):
    xb = x_ref[0]
    cb = cb_ref[...]
    crossT = lax.dot_general(cb, xb, (((1,), (0,)), ((), ())),
                             preferred_element_type=jnp.float32)
    xsq = xsq_ref[0]
    t = (xsq + cbsq_ref[...]) - 2.0 * crossT
    dist = jnp.sqrt(jnp.clip(t, 0.0, None))
    minv = jnp.min(dist, axis=0, keepdims=True)
    kiota = lax.broadcasted_iota(jnp.int32, (_K, _N), 0)
    masked = jnp.where(dist == minv, kiota, _K)
    idx_ref[0] = jnp.min(masked, axis=0, keepdims=True)


def _sc_gather(cbt_flat, idx_flat):
    mesh = plsc.VectorSubcoreMesh(core_axis_name="c", subcore_axis_name="s")

    @functools.partial(
        pl.kernel,
        mesh=mesh,
        out_type=jax.ShapeDtypeStruct((_B * _C * _N,), jnp.float32),
        compiler_params=pltpu.CompilerParams(needs_layout_passes=False),
        scratch_types=[
            pltpu.VMEM((_CPW * _K,), jnp.float32),
            pltpu.VMEM((_B * _N,), jnp.int32),
            pltpu.VMEM((2 * _CPW * _N,), jnp.float32),
            pltpu.SemaphoreType.DMA,
            pltpu.SemaphoreType.DMA,
        ],
    )
    def k(cbt_hbm, idx_hbm, q_hbm, tab_v, idx_v, out_v, sem0, sem1):
        wid = lax.axis_index("s") * 2 + lax.axis_index("c")

        @pl.when(wid < _AW)
        def _():
            cs = wid * _CPW
            pltpu.sync_copy(cbt_hbm.at[pl.ds(cs * _K, _CPW * _K)], tab_v)
            pltpu.sync_copy(idx_hbm, idx_v)
            sems = (sem0, sem1)
            descs = [None, None]
            for b in range(_B):
                slot = b % 2
                if descs[slot] is not None:
                    descs[slot].wait()
                ob = slot * _CPW * _N

                @plsc.parallel_loop(0, _N // _L, unroll=8)
                def body(i):
                    idxs = idx_v[pl.ds(b * _N + i * _L, _L)]
                    for r in range(_CPW):
                        out_v[pl.ds(ob + r * _N + i * _L, _L)] = (
                            plsc.load_gather(tab_v.at[pl.ds(r * _K, _K)],
                                             [idxs]))

                descs[slot] = pltpu.async_copy(
                    out_v.at[pl.ds(ob, _CPW * _N)],
                    q_hbm.at[pl.ds(b * _C * _N + cs * _N, _CPW * _N)],
                    sems[slot])
            descs[0].wait()
            descs[1].wait()

    return k(cbt_flat, idx_flat)


def kernel(x, codebook):
    b, c, h, w = x.shape
    n = h * w
    x3 = x.reshape(b, c, n)
    cb_sq = jnp.sum(codebook * codebook, axis=-1)
    cbsq2 = cb_sq.reshape(_K, 1)
    flat = jnp.transpose(x, (0, 2, 3, 1)).reshape(b, n, c).astype(jnp.float32)
    x_sq = jnp.sum(flat * flat, axis=-1)
    xsq3 = x_sq.reshape(b, 1, n)
    idx = pl.pallas_call(
        _argmin_body,
        grid=(b,),
        in_specs=[
            pl.BlockSpec((1, c, n), lambda i: (i, 0, 0)),
            pl.BlockSpec((_K, c), lambda i: (0, 0)),
            pl.BlockSpec((_K, 1), lambda i: (0, 0)),
            pl.BlockSpec((1, 1, n), lambda i: (i, 0, 0)),
        ],
        out_specs=pl.BlockSpec((1, 1, n), lambda i: (i, 0, 0)),
        out_shape=jax.ShapeDtypeStruct((b, 1, n), jnp.int32),
    )(x3, codebook, cbsq2, xsq3)
    idx2 = idx.reshape(b, n)
    cbt = jnp.transpose(codebook, (1, 0))
    qf = _sc_gather(cbt.reshape(-1), idx2.reshape(-1))
    quantized = qf.reshape(b, c, h, w)
    embed_index = idx2.reshape(b, h, w)
    loss = jnp.array([0.0], dtype=jnp.float32)
    return (quantized, embed_index, loss)
